# speedup vs baseline: 4.9255x; 4.9255x over previous
import jax
import jax.numpy as jnp
from jax import lax
from jax.experimental import pallas as pl
from jax.experimental.pallas import tpu as pltpu
from jax.experimental.pallas import tpu_sc as plsc

B = 4096
V = 1000
D0 = 64 * 64
D1 = 32 * 32
D2 = 16 * 16
DTOT = D0 + D1 + D2

NC = 2
NS = 16
NW = NC * NS
B_PER_W = B // NW
CHUNK = 16
NCHUNK = B_PER_W // CHUNK

_MESH = plsc.VectorSubcoreMesh(core_axis_name="c", subcore_axis_name="s")


def _gather_body(x_hbm, rep0_hbm, rep1_hbm, rep2_hbm, out_hbm,
                 idx_v, rows0_v, rows1_v, rows2_v, sem0, sem1, sem2):
    wid = lax.axis_index("s") * NC + lax.axis_index("c")
    base = wid * B_PER_W
    pltpu.sync_copy(x_hbm.at[pl.ds(base, B_PER_W)], idx_v)
    for c in range(NCHUNK):
        idx_c = idx_v.at[pl.ds(c * CHUNK, CHUNK)]
        cp0 = pltpu.async_copy(rep0_hbm.at[idx_c], rows0_v, sem0)
        cp1 = pltpu.async_copy(rep1_hbm.at[idx_c], rows1_v, sem1)
        cp2 = pltpu.async_copy(rep2_hbm.at[idx_c], rows2_v, sem2)
        row = base + c * CHUNK
        cp0.wait()
        pltpu.sync_copy(rows0_v, out_hbm.at[pl.ds(row, CHUNK), pl.ds(0, D0)])
        cp1.wait()
        pltpu.sync_copy(rows1_v, out_hbm.at[pl.ds(row, CHUNK), pl.ds(D0, D1)])
        cp2.wait()
        pltpu.sync_copy(rows2_v, out_hbm.at[pl.ds(row, CHUNK), pl.ds(D0 + D1, D2)])


_gather_call = pl.kernel(
    _gather_body,
    out_type=jax.ShapeDtypeStruct((B, DTOT), jnp.float32),
    mesh=_MESH,
    scratch_types=[
        pltpu.VMEM((B_PER_W,), jnp.int32),
        pltpu.VMEM((CHUNK, D0), jnp.float32),
        pltpu.VMEM((CHUNK, D1), jnp.float32),
        pltpu.VMEM((CHUNK, D2), jnp.float32),
        pltpu.SemaphoreType.DMA,
        pltpu.SemaphoreType.DMA,
        pltpu.SemaphoreType.DMA,
    ],
)


@jax.jit
def kernel(x, rep0, rep1, rep2):
    x = x.astype(jnp.int32)
    r0 = rep0.reshape(V, D0)
    r1 = rep1.reshape(V, D1)
    r2 = rep2.reshape(V, D2)
    return _gather_call(x, r0, r1, r2)


# speedup vs baseline: 5.1834x; 1.0524x over previous
import jax
import jax.numpy as jnp
from jax import lax
from jax.experimental import pallas as pl
from jax.experimental.pallas import tpu as pltpu
from jax.experimental.pallas import tpu_sc as plsc

B = 4096
V = 1000
D0 = 64 * 64
D1 = 32 * 32
D2 = 16 * 16
DTOT = D0 + D1 + D2
COL = (0, D0, D0 + D1)
DIMS = (D0, D1, D2)

NC = 2
NS = 16
NW = NC * NS
B_PER_W = B // NW
CHUNK = 8
NCHUNK = B_PER_W // CHUNK

_MESH = plsc.VectorSubcoreMesh(core_axis_name="c", subcore_axis_name="s")


def _gather_body(x_hbm, rep0_hbm, rep1_hbm, rep2_hbm, out_hbm,
                 idx_v,
                 r0a, r1a, r2a, r0b, r1b, r2b,
                 g0a, g1a, g2a, g0b, g1b, g2b,
                 wa, wb):
    reps = (rep0_hbm, rep1_hbm, rep2_hbm)
    bufs = ((r0a, r1a, r2a), (r0b, r1b, r2b))
    gsems = ((g0a, g1a, g2a), (g0b, g1b, g2b))
    wsems = (wa, wb)

    wid = lax.axis_index("s") * NC + lax.axis_index("c")
    base = wid * B_PER_W
    pltpu.sync_copy(x_hbm.at[pl.ds(base, B_PER_W)], idx_v)

    def issue_gathers(c, slot):
        idx_c = idx_v.at[pl.ds(c * CHUNK, CHUNK)]
        return [pltpu.async_copy(reps[t].at[idx_c], bufs[slot][t], gsems[slot][t])
                for t in range(3)]

    gathers = [issue_gathers(0, 0), None]
    pend_writes = [None, None]
    for c in range(NCHUNK):
        slot = c & 1
        if c + 1 < NCHUNK:
            other = 1 - slot
            if pend_writes[other] is not None:
                for w in pend_writes[other]:
                    w.wait()
                pend_writes[other] = None
            gathers[other] = issue_gathers(c + 1, other)
        row = base + c * CHUNK
        ws = []
        for t in range(3):
            gathers[slot][t].wait()
            ws.append(pltpu.async_copy(
                bufs[slot][t],
                out_hbm.at[pl.ds(row, CHUNK), pl.ds(COL[t], DIMS[t])],
                wsems[slot]))
        pend_writes[slot] = ws
    for slot in (0, 1):
        if pend_writes[slot] is not None:
            for w in pend_writes[slot]:
                w.wait()


_gather_call = pl.kernel(
    _gather_body,
    out_type=jax.ShapeDtypeStruct((B, DTOT), jnp.float32),
    mesh=_MESH,
    scratch_types=[
        pltpu.VMEM((B_PER_W,), jnp.int32),
        pltpu.VMEM((CHUNK, D0), jnp.float32),
        pltpu.VMEM((CHUNK, D1), jnp.float32),
        pltpu.VMEM((CHUNK, D2), jnp.float32),
        pltpu.VMEM((CHUNK, D0), jnp.float32),
        pltpu.VMEM((CHUNK, D1), jnp.float32),
        pltpu.VMEM((CHUNK, D2), jnp.float32),
        pltpu.SemaphoreType.DMA,
        pltpu.SemaphoreType.DMA,
        pltpu.SemaphoreType.DMA,
        pltpu.SemaphoreType.DMA,
        pltpu.SemaphoreType.DMA,
        pltpu.SemaphoreType.DMA,
        pltpu.SemaphoreType.DMA,
        pltpu.SemaphoreType.DMA,
    ],
)


@jax.jit
def kernel(x, rep0, rep1, rep2):
    x = x.astype(jnp.int32)
    r0 = rep0.reshape(V, D0)
    r1 = rep1.reshape(V, D1)
    r2 = rep2.reshape(V, D2)
    return _gather_call(x, r0, r1, r2)


# speedup vs baseline: 5.2059x; 1.0043x over previous
import jax
import jax.numpy as jnp
from jax import lax
from jax.experimental import pallas as pl
from jax.experimental.pallas import tpu as pltpu
from jax.experimental.pallas import tpu_sc as plsc

B = 4096
V = 1000
D0 = 64 * 64
D1 = 32 * 32
D2 = 16 * 16
DTOT = D0 + D1 + D2
COL = (0, D0, D0 + D1)
DIMS = (D0, D1, D2)

NC = 2
NS = 16
NW = NC * NS
B_PER_W = B // NW
CHUNK = 8
NCHUNK = B_PER_W // CHUNK

_MESH = plsc.VectorSubcoreMesh(core_axis_name="c", subcore_axis_name="s")


def _gather_body(x_hbm, rep0_hbm, rep1_hbm, rep2_hbm, out_hbm,
                 idx_v, fat_a, fat_b, gsem_a, gsem_b, wsem_a, wsem_b):
    reps = (rep0_hbm, rep1_hbm, rep2_hbm)
    bufs = (fat_a, fat_b)
    gsems = (gsem_a, gsem_b)
    wsems = (wsem_a, wsem_b)

    wid = lax.axis_index("s") * NC + lax.axis_index("c")
    base = wid * B_PER_W
    pltpu.sync_copy(x_hbm.at[pl.ds(base, B_PER_W)], idx_v)

    def issue_gathers(c, slot):
        idx_c = idx_v.at[pl.ds(c * CHUNK, CHUNK)]
        return [pltpu.async_copy(
                    reps[t].at[idx_c],
                    bufs[slot].at[:, pl.ds(COL[t], DIMS[t])],
                    gsems[slot])
                for t in range(3)]

    gathers = [issue_gathers(0, 0), None]
    pend_write = [None, None]
    for c in range(NCHUNK):
        slot = c & 1
        if c + 1 < NCHUNK:
            other = 1 - slot
            if pend_write[other] is not None:
                pend_write[other].wait()
                pend_write[other] = None
            gathers[other] = issue_gathers(c + 1, other)
        for g in gathers[slot]:
            g.wait()
        row = base + c * CHUNK
        pend_write[slot] = pltpu.async_copy(
            bufs[slot], out_hbm.at[pl.ds(row, CHUNK)], wsems[slot])
    for slot in (0, 1):
        if pend_write[slot] is not None:
            pend_write[slot].wait()


_gather_call = pl.kernel(
    _gather_body,
    out_type=jax.ShapeDtypeStruct((B, DTOT), jnp.float32),
    mesh=_MESH,
    scratch_types=[
        pltpu.VMEM((B_PER_W,), jnp.int32),
        pltpu.VMEM((CHUNK, DTOT), jnp.float32),
        pltpu.VMEM((CHUNK, DTOT), jnp.float32),
        pltpu.SemaphoreType.DMA,
        pltpu.SemaphoreType.DMA,
        pltpu.SemaphoreType.DMA,
        pltpu.SemaphoreType.DMA,
    ],
)


@jax.jit
def kernel(x, rep0, rep1, rep2):
    x = x.astype(jnp.int32)
    r0 = rep0.reshape(V, D0)
    r1 = rep1.reshape(V, D1)
    r2 = rep2.reshape(V, D2)
    return _gather_call(x, r0, r1, r2)


# speedup vs baseline: 5.2469x; 1.0079x over previous
import jax
import jax.numpy as jnp
from jax import lax
from jax.experimental import pallas as pl
from jax.experimental.pallas import tpu as pltpu
from jax.experimental.pallas import tpu_sc as plsc

B = 4096
V = 1000
D0 = 64 * 64
D1 = 32 * 32
D2 = 16 * 16
DTOT = D0 + D1 + D2
COL = (0, D0, D0 + D1)
DIMS = (D0, D1, D2)

NC = 2
NS = 16
NW = NC * NS
B_PER_W = B // NW
CHUNK = 8
NCHUNK = B_PER_W // CHUNK

_MESH = plsc.VectorSubcoreMesh(core_axis_name="c", subcore_axis_name="s")


def _gather_body(x_hbm, rep0_hbm, rep1_hbm, rep2_hbm, out_hbm,
                 idx_sh, idx_s, shared, gsem_a, gsem_b, wsem_a, wsem_b):
    reps = (rep0_hbm, rep1_hbm, rep2_hbm)
    gsems = (gsem_a, gsem_b)
    wsems = (wsem_a, wsem_b)

    sid = lax.axis_index("s")
    wid = sid * NC + lax.axis_index("c")
    base = wid * B_PER_W
    pltpu.sync_copy(x_hbm.at[pl.ds(base, B_PER_W)], idx_sh.at[sid])
    pltpu.sync_copy(idx_sh.at[sid], idx_s)

    def issue_gathers(c, slot):
        cps = []
        for r in range(CHUNK):
            ival = idx_s[c * CHUNK + r]
            for t in range(3):
                cps.append(pltpu.async_copy(
                    reps[t].at[ival],
                    shared.at[sid, slot, r, pl.ds(COL[t], DIMS[t])],
                    gsems[slot]))
        return cps

    gathers = [issue_gathers(0, 0), None]
    pend_write = [None, None]
    for c in range(NCHUNK):
        slot = c & 1
        if c + 1 < NCHUNK:
            other = 1 - slot
            if pend_write[other] is not None:
                pend_write[other].wait()
                pend_write[other] = None
            gathers[other] = issue_gathers(c + 1, other)
        for g in gathers[slot]:
            g.wait()
        row = base + c * CHUNK
        pend_write[slot] = pltpu.async_copy(
            shared.at[sid, slot], out_hbm.at[pl.ds(row, CHUNK)], wsems[slot])
    for slot in (0, 1):
        if pend_write[slot] is not None:
            pend_write[slot].wait()


_gather_call = pl.kernel(
    _gather_body,
    out_type=jax.ShapeDtypeStruct((B, DTOT), jnp.float32),
    mesh=_MESH,
    scratch_types=[
        pltpu.VMEM_SHARED((NS, B_PER_W), jnp.int32),
        pltpu.SMEM((B_PER_W,), jnp.int32),
        pltpu.VMEM_SHARED((NS, 2, CHUNK, DTOT), jnp.float32),
        pltpu.SemaphoreType.DMA,
        pltpu.SemaphoreType.DMA,
        pltpu.SemaphoreType.DMA,
        pltpu.SemaphoreType.DMA,
    ],
)


@jax.jit
def kernel(x, rep0, rep1, rep2):
    x = x.astype(jnp.int32)
    r0 = rep0.reshape(V, D0)
    r1 = rep1.reshape(V, D1)
    r2 = rep2.reshape(V, D2)
    return _gather_call(x, r0, r1, r2)
